# Initial kernel scaffold; baseline (speedup 1.0000x reference)
import jax
import jax.numpy as jnp
from jax.experimental import pallas as pl


def kernel(x, edge_index, W1, a_src1, a_dst1, b1, W2, a_src2, a_dst2, b2):
    raise NotImplementedError("write your pallas kernel here")


# speedup vs baseline: 28.2439x; 28.2439x over previous
import functools

import jax
import jax.numpy as jnp
from jax import lax
from jax.experimental import pallas as pl
from jax.experimental.pallas import tpu as pltpu
from jax.experimental.pallas import tpu_sc as plsc

N = 10000
E = 320000
IN = 128
HID = 16
HEADS = 8
OUT = 64

NP = 10240
BM = 256
GRID = NP // BM

NC = 2
NS = 16
L = 16
B = 128
NCHUNK = E // B
CPT_HALF = -(-NCHUNK // NS)
CPT_ALL = -(-NCHUNK // (NC * NS))
P = 5120
OWN = 5000
ROWS1 = P // NS
ROWS2 = NP // NS

_mesh = plsc.VectorSubcoreMesh(
    core_axis_name="c", subcore_axis_name="s", num_cores=NC, num_subcores=NS)


def _tc1_body(x_ref, w_ref, a_ref, h_ref, sa_ref, sb_ref, mx_ref):
    h = jnp.dot(x_ref[...], w_ref[...], preferred_element_type=jnp.float32)
    h_ref[...] = h
    s = jnp.dot(h, a_ref[...], preferred_element_type=jnp.float32)
    sa_ref[...] = s
    sb_ref[...] = jnp.concatenate([s[:, 8:], s[:, :8]], axis=1)
    m = jnp.max(s, axis=0, keepdims=True)

    @pl.when(pl.program_id(0) == 0)
    def _():
        mx_ref[...] = m

    @pl.when(pl.program_id(0) != 0)
    def _():
        mx_ref[...] = jnp.maximum(mx_ref[...], m)


def _tc1(x_pad, W1, A1):
    return pl.pallas_call(
        _tc1_body,
        grid=(GRID,),
        in_specs=[
            pl.BlockSpec((BM, IN), lambda i: (i, 0)),
            pl.BlockSpec((IN, HEADS * HID), lambda i: (0, 0)),
            pl.BlockSpec((HEADS * HID, 16), lambda i: (0, 0)),
        ],
        out_specs=[
            pl.BlockSpec((BM, HEADS * HID), lambda i: (i, 0)),
            pl.BlockSpec((BM, 16), lambda i: (i, 0)),
            pl.BlockSpec((BM, 16), lambda i: (i, 0)),
            pl.BlockSpec((1, 16), lambda i: (0, 0)),
        ],
        out_shape=[
            jax.ShapeDtypeStruct((NP, HEADS * HID), jnp.float32),
            jax.ShapeDtypeStruct((NP, 16), jnp.float32),
            jax.ShapeDtypeStruct((NP, 16), jnp.float32),
            jax.ShapeDtypeStruct((1, 16), jnp.float32),
        ],
    )(x_pad, W1, A1)


def _tc2_body(p0_ref, p1_ref, b_ref, w_ref, a_ref, h_ref, sa_ref, mx_ref):
    t = jnp.maximum(p0_ref[...] + p1_ref[...] + b_ref[...], 0.0)
    h = jnp.dot(t, w_ref[...], preferred_element_type=jnp.float32)
    h_ref[...] = h
    s = jnp.dot(h, a_ref[...], preferred_element_type=jnp.float32)
    sa_ref[...] = s
    m = jnp.max(s, axis=0, keepdims=True)

    @pl.when(pl.program_id(0) == 0)
    def _():
        mx_ref[...] = m

    @pl.when(pl.program_id(0) != 0)
    def _():
        mx_ref[...] = jnp.maximum(mx_ref[...], m)


def _tc2(p0, p1, b1r, W2, A2):
    return pl.pallas_call(
        _tc2_body,
        grid=(GRID,),
        in_specs=[
            pl.BlockSpec((BM, HEADS * HID), lambda i: (i, 0)),
            pl.BlockSpec((BM, HEADS * HID), lambda i: (i, 0)),
            pl.BlockSpec((1, HEADS * HID), lambda i: (0, 0)),
            pl.BlockSpec((HEADS * HID, OUT), lambda i: (0, 0)),
            pl.BlockSpec((OUT, 16), lambda i: (0, 0)),
        ],
        out_specs=[
            pl.BlockSpec((BM, OUT), lambda i: (i, 0)),
            pl.BlockSpec((BM, 16), lambda i: (i, 0)),
            pl.BlockSpec((1, 16), lambda i: (0, 0)),
        ],
        out_shape=[
            jax.ShapeDtypeStruct((NP, OUT), jnp.float32),
            jax.ShapeDtypeStruct((NP, 16), jnp.float32),
            jax.ShapeDtypeStruct((1, 16), jnp.float32),
        ],
    )(p0, p1, b1r, W2, A2)


def _tc3_body(q0_ref, q1_ref, b_ref, o_ref):
    o = q0_ref[...] + q1_ref[...] + b_ref[...]
    m = jnp.max(o, axis=1, keepdims=True)
    z = o - m
    lse = jnp.log(jnp.sum(jnp.exp(z), axis=1, keepdims=True))
    o_ref[...] = z - lse


def _tc3(q0, q1, b2r):
    return pl.pallas_call(
        _tc3_body,
        grid=(GRID,),
        in_specs=[
            pl.BlockSpec((BM, OUT), lambda i: (i, 0)),
            pl.BlockSpec((BM, OUT), lambda i: (i, 0)),
            pl.BlockSpec((1, OUT), lambda i: (0, 0)),
        ],
        out_specs=pl.BlockSpec((BM, OUT), lambda i: (i, 0)),
        out_shape=jax.ShapeDtypeStruct((NP, OUT), jnp.float32),
    )(q0, q1, b2r)


@functools.partial(
    pl.kernel,
    out_type=jax.ShapeDtypeStruct((NC * P, 16), jnp.float32),
    mesh=_mesh,
    compiler_params=pltpu.CompilerParams(use_tc_tiling_on_sc=False, needs_layout_passes=False),
    scratch_types=[
        pltpu.VMEM((B,), jnp.int32),
        pltpu.VMEM((B,), jnp.int32),
        pltpu.VMEM((B,), jnp.int32),
        pltpu.VMEM((B, 16), jnp.float32),
        pltpu.VMEM((B, 16), jnp.float32),
        pltpu.VMEM((B, 16), jnp.float32),
        pltpu.VMEM((L,), jnp.float32),
        pltpu.VMEM_SHARED((P, 16), jnp.float32),
        pltpu.SemaphoreType.DMA,
        pltpu.SemaphoreType.DMA,
    ],
)
def _sc1_pass1(src_hbm, dst_hbm, ta_hbm, tb_hbm, k_hbm, z_hbm, den_hbm,
               src_v, dst_v, sc_v, sa_v, sb_v, ex_v, k_v, den_sp,
               sem0, sem1):
    c = lax.axis_index("c")
    s = lax.axis_index("s")
    pltpu.sync_copy(k_hbm, k_v)
    kv = k_v[...]
    pltpu.sync_copy(z_hbm.at[pl.ds(s * ROWS1, ROWS1), :],
                    den_sp.at[pl.ds(s * ROWS1, ROWS1), :])
    plsc.subcore_barrier()
    base = c * OWN

    def chunk(j, carry):
        ch = s * CPT_HALF + j

        @pl.when(ch < NCHUNK)
        def _():
            off = ch * B
            pltpu.sync_copy(src_hbm.at[pl.ds(off, B)], src_v)
            pltpu.sync_copy(dst_hbm.at[pl.ds(off, B)], dst_v)
            ga = pltpu.async_copy(ta_hbm.at[src_v], sa_v, sem0)
            gb = pltpu.async_copy(tb_hbm.at[dst_v], sb_v, sem1)
            ga.wait()
            gb.wait()

            def edge(i, carry2):
                e = sa_v[i, :] + sb_v[i, :]
                e = jnp.maximum(e, 0.2 * e)
                ex_v[i, :] = jnp.exp(e - kv)
                return carry2

            lax.fori_loop(0, B, edge, 0)

            def vmap8(j2, carry2):
                d = dst_v[pl.ds(j2 * L, L)]
                dl = d - base
                ok = (dl >= 0) & (dl < OWN)
                sc_v[pl.ds(j2 * L, L)] = jnp.where(ok, dl, OWN)
                return carry2

            lax.fori_loop(0, B // L, vmap8, 0)
            pltpu.sync_copy(ex_v, den_sp.at[sc_v], add=True)

        return carry

    lax.fori_loop(0, CPT_HALF, chunk, 0)
    plsc.subcore_barrier()
    pltpu.sync_copy(den_sp.at[pl.ds(s * ROWS1, ROWS1), :],
                    den_hbm.at[pl.ds(c * P + s * ROWS1, ROWS1), :])


@functools.partial(
    pl.kernel,
    out_type=jax.ShapeDtypeStruct((NC, NP, HEADS * HID), jnp.float32),
    mesh=_mesh,
    compiler_params=pltpu.CompilerParams(use_tc_tiling_on_sc=False, needs_layout_passes=False),
    scratch_types=[
        pltpu.VMEM((B,), jnp.int32),
        pltpu.VMEM((B,), jnp.int32),
        pltpu.VMEM((B,), jnp.int32),
        pltpu.VMEM((B, 16), jnp.float32),
        pltpu.VMEM((B, 16), jnp.float32),
        pltpu.VMEM((B, 16), jnp.float32),
        pltpu.VMEM((B, 16), jnp.float32),
        pltpu.VMEM((B, HEADS * HID), jnp.float32),
        pltpu.VMEM((B, HEADS * HID), jnp.float32),
        pltpu.VMEM((L,), jnp.float32),
        pltpu.VMEM_SHARED((NP, HEADS * HID), jnp.float32),
        pltpu.SemaphoreType.DMA,
        pltpu.SemaphoreType.DMA,
        pltpu.SemaphoreType.DMA,
        pltpu.SemaphoreType.DMA,
    ],
)
def _sc1_pass2(src_hbm, dst_hbm, ta_hbm, tb_hbm, k_hbm, den_hbm, h_hbm,
               z_hbm, out_hbm,
               src_v, dst_v, gi_v, sa_v, sb_v, dn_v, al_v, h_v, msg_v, k_v,
               out_sp, sem0, sem1, sem2, sem3):
    c = lax.axis_index("c")
    s = lax.axis_index("s")
    wid = c * NS + s
    pltpu.sync_copy(k_hbm, k_v)
    kv = k_v[...]
    pltpu.sync_copy(z_hbm.at[pl.ds(s * ROWS2, ROWS2), :],
                    out_sp.at[pl.ds(s * ROWS2, ROWS2), :])
    plsc.subcore_barrier()

    def chunk(j, carry):
        ch = wid * CPT_ALL + j

        @pl.when(ch < NCHUNK)
        def _():
            off = ch * B
            pltpu.sync_copy(src_hbm.at[pl.ds(off, B)], src_v)
            pltpu.sync_copy(dst_hbm.at[pl.ds(off, B)], dst_v)

            def vmap8(j2, carry2):
                d = dst_v[pl.ds(j2 * L, L)]
                gi_v[pl.ds(j2 * L, L)] = jnp.where(d >= OWN, d + (P - OWN), d)
                return carry2

            lax.fori_loop(0, B // L, vmap8, 0)
            ga = pltpu.async_copy(ta_hbm.at[src_v], sa_v, sem0)
            gb = pltpu.async_copy(tb_hbm.at[dst_v], sb_v, sem1)
            gd = pltpu.async_copy(den_hbm.at[gi_v], dn_v, sem2)
            gh = pltpu.async_copy(h_hbm.at[src_v], h_v, sem3)
            ga.wait()
            gb.wait()
            gd.wait()

            def edge_a(i, carry2):
                e = sa_v[i, :] + sb_v[i, :]
                e = jnp.maximum(e, 0.2 * e)
                ex = jnp.exp(e - kv)
                al_v[i, :] = ex / (dn_v[i, :] + 1e-16)
                return carry2

            lax.fori_loop(0, B, edge_a, 0)
            gh.wait()

            def edge_m(i, carry2):
                row_i = jnp.full((L,), i, jnp.int32)
                for hd in range(HEADS):
                    av = plsc.load_gather(
                        al_v, [row_i, jnp.full((L,), hd, jnp.int32)])
                    msg_v[i, pl.ds(hd * HID, HID)] = (
                        h_v[i, pl.ds(hd * HID, HID)] * av)
                return carry2

            lax.fori_loop(0, B, edge_m, 0)
            pltpu.sync_copy(msg_v, out_sp.at[dst_v], add=True)

        return carry

    lax.fori_loop(0, CPT_ALL, chunk, 0)
    plsc.subcore_barrier()
    pltpu.sync_copy(out_sp.at[pl.ds(s * ROWS2, ROWS2), :],
                    out_hbm.at[c, pl.ds(s * ROWS2, ROWS2), :])


@functools.partial(
    pl.kernel,
    out_type=jax.ShapeDtypeStruct((NC * P,), jnp.float32),
    mesh=_mesh,
    compiler_params=pltpu.CompilerParams(use_tc_tiling_on_sc=False, needs_layout_passes=False),
    scratch_types=[
        pltpu.VMEM((B,), jnp.int32),
        pltpu.VMEM((B,), jnp.int32),
        pltpu.VMEM((B,), jnp.int32),
        pltpu.VMEM((B,), jnp.float32),
        pltpu.VMEM((NP,), jnp.float32),
        pltpu.VMEM((NP,), jnp.float32),
        pltpu.VMEM((L,), jnp.float32),
        pltpu.VMEM_SHARED((P,), jnp.float32),
    ],
)
def _sc2_pass1(src_hbm, dst_hbm, as_hbm, ad_hbm, k_hbm, z_hbm, den_hbm,
               src_v, dst_v, sc_v, ex_v, as_v, ad_v, k_v, den_sp):
    c = lax.axis_index("c")
    s = lax.axis_index("s")
    pltpu.sync_copy(k_hbm, k_v)
    kv = k_v[...]
    pltpu.sync_copy(as_hbm, as_v)
    pltpu.sync_copy(ad_hbm, ad_v)
    pltpu.sync_copy(z_hbm.at[pl.ds(s * ROWS1, ROWS1)],
                    den_sp.at[pl.ds(s * ROWS1, ROWS1)])
    plsc.subcore_barrier()
    base = c * OWN

    def chunk(j, carry):
        ch = s * CPT_HALF + j

        @pl.when(ch < NCHUNK)
        def _():
            off = ch * B
            pltpu.sync_copy(src_hbm.at[pl.ds(off, B)], src_v)
            pltpu.sync_copy(dst_hbm.at[pl.ds(off, B)], dst_v)

            def vmap8(j2, carry2):
                sl = pl.ds(j2 * L, L)
                sv = plsc.load_gather(as_v, [src_v[sl]])
                dv = plsc.load_gather(ad_v, [dst_v[sl]])
                e = sv + dv
                e = jnp.maximum(e, 0.2 * e)
                ex_v[sl] = jnp.exp(e - kv)
                d = dst_v[sl]
                dl = d - base
                ok = (dl >= 0) & (dl < OWN)
                sc_v[sl] = jnp.where(ok, dl, OWN)
                return carry2

            lax.fori_loop(0, B // L, vmap8, 0)
            pltpu.sync_copy(ex_v, den_sp.at[sc_v], add=True)

        return carry

    lax.fori_loop(0, CPT_HALF, chunk, 0)
    plsc.subcore_barrier()
    pltpu.sync_copy(den_sp.at[pl.ds(s * ROWS1, ROWS1)],
                    den_hbm.at[pl.ds(c * P + s * ROWS1, ROWS1)])


@functools.partial(
    pl.kernel,
    out_type=jax.ShapeDtypeStruct((NC, NP, OUT), jnp.float32),
    mesh=_mesh,
    compiler_params=pltpu.CompilerParams(use_tc_tiling_on_sc=False, needs_layout_passes=False),
    scratch_types=[
        pltpu.VMEM((B,), jnp.int32),
        pltpu.VMEM((B,), jnp.int32),
        pltpu.VMEM((B,), jnp.float32),
        pltpu.VMEM((NP,), jnp.float32),
        pltpu.VMEM((NP,), jnp.float32),
        pltpu.VMEM((NP,), jnp.float32),
        pltpu.VMEM((B, OUT), jnp.float32),
        pltpu.VMEM((B, OUT), jnp.float32),
        pltpu.VMEM((L,), jnp.float32),
        pltpu.VMEM_SHARED((NP, OUT), jnp.float32),
        pltpu.SemaphoreType.DMA,
    ],
)
def _sc2_pass2(src_hbm, dst_hbm, as_hbm, ad_hbm, k_hbm, den_hbm, h_hbm,
               z_hbm, out_hbm,
               src_v, dst_v, al_v, as_v, ad_v, dn_v, h_v, msg_v, k_v,
               out_sp, sem0):
    c = lax.axis_index("c")
    s = lax.axis_index("s")
    wid = c * NS + s
    pltpu.sync_copy(k_hbm, k_v)
    kv = k_v[...]
    pltpu.sync_copy(as_hbm, as_v)
    pltpu.sync_copy(ad_hbm, ad_v)
    pltpu.sync_copy(den_hbm.at[pl.ds(0, NP)], dn_v)
    pltpu.sync_copy(z_hbm.at[pl.ds(s * ROWS2, ROWS2), :],
                    out_sp.at[pl.ds(s * ROWS2, ROWS2), :])
    plsc.subcore_barrier()

    def chunk(j, carry):
        ch = wid * CPT_ALL + j

        @pl.when(ch < NCHUNK)
        def _():
            off = ch * B
            pltpu.sync_copy(src_hbm.at[pl.ds(off, B)], src_v)
            pltpu.sync_copy(dst_hbm.at[pl.ds(off, B)], dst_v)
            gh = pltpu.async_copy(h_hbm.at[src_v], h_v, sem0)

            def vmap8(j2, carry2):
                sl = pl.ds(j2 * L, L)
                sv = plsc.load_gather(as_v, [src_v[sl]])
                d = dst_v[sl]
                dv = plsc.load_gather(ad_v, [d])
                e = sv + dv
                e = jnp.maximum(e, 0.2 * e)
                ex = jnp.exp(e - kv)
                gi = jnp.where(d >= OWN, d + (P - OWN), d)
                den = plsc.load_gather(dn_v, [gi])
                al_v[sl] = ex / (den + 1e-16)
                return carry2

            lax.fori_loop(0, B // L, vmap8, 0)
            gh.wait()

            def edge_m(i, carry2):
                av = plsc.load_gather(al_v, [jnp.full((L,), i, jnp.int32)])
                for k in range(OUT // HID):
                    msg_v[i, pl.ds(k * HID, HID)] = (
                        h_v[i, pl.ds(k * HID, HID)] * av)
                return carry2

            lax.fori_loop(0, B, edge_m, 0)
            pltpu.sync_copy(msg_v, out_sp.at[dst_v], add=True)

        return carry

    lax.fori_loop(0, CPT_ALL, chunk, 0)
    plsc.subcore_barrier()
    pltpu.sync_copy(out_sp.at[pl.ds(s * ROWS2, ROWS2), :],
                    out_hbm.at[c, pl.ds(s * ROWS2, ROWS2), :])


def kernel(x, edge_index, W1, a_src1, a_dst1, b1, W2, a_src2, a_dst2, b2):
    x_pad = jnp.pad(x, ((0, NP - N), (0, 0)))
    src = edge_index[0].astype(jnp.int32)
    dst = edge_index[1].astype(jnp.int32)

    eye = jnp.repeat(jnp.eye(HEADS, dtype=jnp.float32), HID, axis=0)
    A_src = eye * a_src1.reshape(-1)[:, None]
    A_dst = eye * a_dst1.reshape(-1)[:, None]
    A1 = jnp.concatenate([A_src, A_dst], axis=1)

    h1, ta1, tb1, mx1 = _tc1(x_pad, W1, A1)
    k1 = mx1[0, :8] + mx1[0, 8:]
    k1 = jnp.maximum(k1, 0.2 * k1)
    k16_1 = jnp.concatenate([k1, k1])

    z16 = jnp.zeros((P, 16), jnp.float32)
    z128 = jnp.zeros((NP, HEADS * HID), jnp.float32)
    den1 = _sc1_pass1(src, dst, ta1, tb1, k16_1, z16)
    out1p = _sc1_pass2(src, dst, ta1, tb1, k16_1, den1, h1, z128)

    A2 = jnp.zeros((OUT, 16), jnp.float32)
    A2 = A2.at[:, 0].set(a_src2[0]).at[:, 1].set(a_dst2[0])
    h2, sa2, mx2 = _tc2(out1p[0], out1p[1], b1.reshape(1, -1), W2, A2)
    k2 = mx2[0, 0] + mx2[0, 1]
    k2 = jnp.maximum(k2, 0.2 * k2)
    k16_2 = jnp.full((L,), k2, jnp.float32)
    as2 = sa2[:, 0]
    ad2 = sa2[:, 1]

    z1d = jnp.zeros((P,), jnp.float32)
    z64 = jnp.zeros((NP, OUT), jnp.float32)
    den2 = _sc2_pass1(src, dst, as2, ad2, k16_2, z1d)
    out2p = _sc2_pass2(src, dst, as2, ad2, k16_2, den2, h2, z64)

    o = _tc3(out2p[0], out2p[1], b2.reshape(1, -1))
    return o[:N]


# speedup vs baseline: 49.3374x; 1.7468x over previous
import functools

import jax
import jax.numpy as jnp
from jax import lax
from jax.experimental import pallas as pl
from jax.experimental.pallas import tpu as pltpu
from jax.experimental.pallas import tpu_sc as plsc

N = 10000
E = 320000
IN = 128
HID = 16
HEADS = 8
OUT = 64

NP = 10240
BM = 256
GRID = NP // BM

NC = 2
NS = 16
L = 16
B = 128
NCHUNK = E // B
CPT = -(-NCHUNK // (NC * NS))
ROWS = NP // NS

_mesh = plsc.VectorSubcoreMesh(
    core_axis_name="c", subcore_axis_name="s", num_cores=NC, num_subcores=NS)
_scparams = pltpu.CompilerParams(
    use_tc_tiling_on_sc=False, needs_layout_passes=False)


def _bcast(v, idx):
    dn = lax.GatherDimensionNumbers(
        offset_dims=(), collapsed_slice_dims=(0,), start_index_map=(0,))
    return lax.gather(v, idx[:, None], dn, (1,),
                      mode=lax.GatherScatterMode.PROMISE_IN_BOUNDS)


def _tc1_body(x_ref, w_ref, a_ref, h_ref, sa_ref, sb_ref, mx_ref):
    h = jnp.dot(x_ref[...], w_ref[...], preferred_element_type=jnp.float32)
    h_ref[...] = h
    s = jnp.dot(h, a_ref[...], preferred_element_type=jnp.float32)
    sa_ref[...] = s
    sb_ref[...] = jnp.concatenate([s[:, 8:], s[:, :8]], axis=1)
    m = jnp.max(s, axis=0, keepdims=True)

    @pl.when(pl.program_id(0) == 0)
    def _():
        mx_ref[...] = m

    @pl.when(pl.program_id(0) != 0)
    def _():
        mx_ref[...] = jnp.maximum(mx_ref[...], m)


def _tc1(x_pad, W1, A1):
    return pl.pallas_call(
        _tc1_body,
        grid=(GRID,),
        in_specs=[
            pl.BlockSpec((BM, IN), lambda i: (i, 0)),
            pl.BlockSpec((IN, HEADS * HID), lambda i: (0, 0)),
            pl.BlockSpec((HEADS * HID, 16), lambda i: (0, 0)),
        ],
        out_specs=[
            pl.BlockSpec((BM, HEADS * HID), lambda i: (i, 0)),
            pl.BlockSpec((BM, 16), lambda i: (i, 0)),
            pl.BlockSpec((BM, 16), lambda i: (i, 0)),
            pl.BlockSpec((1, 16), lambda i: (0, 0)),
        ],
        out_shape=[
            jax.ShapeDtypeStruct((NP, HEADS * HID), jnp.float32),
            jax.ShapeDtypeStruct((NP, 16), jnp.float32),
            jax.ShapeDtypeStruct((NP, 16), jnp.float32),
            jax.ShapeDtypeStruct((1, 16), jnp.float32),
        ],
    )(x_pad, W1, A1)


def _tc2_body(p0_ref, p1_ref, b_ref, w_ref, a_ref, h_ref, sa_ref, mx_ref):
    t = jnp.maximum(p0_ref[...] + p1_ref[...] + b_ref[...], 0.0)
    h = jnp.dot(t, w_ref[...], preferred_element_type=jnp.float32)
    h_ref[...] = h
    s = jnp.dot(h, a_ref[...], preferred_element_type=jnp.float32)
    sa_ref[...] = s
    m = jnp.max(s, axis=0, keepdims=True)

    @pl.when(pl.program_id(0) == 0)
    def _():
        mx_ref[...] = m

    @pl.when(pl.program_id(0) != 0)
    def _():
        mx_ref[...] = jnp.maximum(mx_ref[...], m)


def _tc2(p0, p1, b1r, W2, A2):
    return pl.pallas_call(
        _tc2_body,
        grid=(GRID,),
        in_specs=[
            pl.BlockSpec((BM, HEADS * HID), lambda i: (i, 0)),
            pl.BlockSpec((BM, HEADS * HID), lambda i: (i, 0)),
            pl.BlockSpec((1, HEADS * HID), lambda i: (0, 0)),
            pl.BlockSpec((HEADS * HID, OUT), lambda i: (0, 0)),
            pl.BlockSpec((OUT, 16), lambda i: (0, 0)),
        ],
        out_specs=[
            pl.BlockSpec((BM, OUT), lambda i: (i, 0)),
            pl.BlockSpec((BM, 16), lambda i: (i, 0)),
            pl.BlockSpec((1, 16), lambda i: (0, 0)),
        ],
        out_shape=[
            jax.ShapeDtypeStruct((NP, OUT), jnp.float32),
            jax.ShapeDtypeStruct((NP, 16), jnp.float32),
            jax.ShapeDtypeStruct((1, 16), jnp.float32),
        ],
    )(p0, p1, b1r, W2, A2)


def _tc3_body(q0_ref, q1_ref, b_ref, o_ref):
    o = q0_ref[...] + q1_ref[...] + b_ref[...]
    m = jnp.max(o, axis=1, keepdims=True)
    z = o - m
    lse = jnp.log(jnp.sum(jnp.exp(z), axis=1, keepdims=True))
    o_ref[...] = z - lse


def _tc3(q0, q1, b2r):
    return pl.pallas_call(
        _tc3_body,
        grid=(GRID,),
        in_specs=[
            pl.BlockSpec((BM, OUT), lambda i: (i, 0)),
            pl.BlockSpec((BM, OUT), lambda i: (i, 0)),
            pl.BlockSpec((1, OUT), lambda i: (0, 0)),
        ],
        out_specs=pl.BlockSpec((BM, OUT), lambda i: (i, 0)),
        out_shape=jax.ShapeDtypeStruct((NP, OUT), jnp.float32),
    )(q0, q1, b2r)


def _tcadd_body(a_ref, b_ref, o_ref):
    o_ref[...] = a_ref[...] + b_ref[...]


def _tc_add(a, b):
    rows, cols = a.shape
    bm = min(rows, BM)
    return pl.pallas_call(
        _tcadd_body,
        grid=(rows // bm,),
        in_specs=[
            pl.BlockSpec((bm, cols), lambda i: (i, 0)),
            pl.BlockSpec((bm, cols), lambda i: (i, 0)),
        ],
        out_specs=pl.BlockSpec((bm, cols), lambda i: (i, 0)),
        out_shape=jax.ShapeDtypeStruct((rows, cols), jnp.float32),
    )(a, b)


@functools.partial(
    pl.kernel,
    out_type=jax.ShapeDtypeStruct((NC, NP, 16), jnp.float32),
    mesh=_mesh,
    compiler_params=_scparams,
    scratch_types=[
        pltpu.VMEM((B,), jnp.int32),
        pltpu.VMEM((B,), jnp.int32),
        pltpu.VMEM((B, 16), jnp.float32),
        pltpu.VMEM((B, 16), jnp.float32),
        pltpu.VMEM((B, 16), jnp.float32),
        pltpu.VMEM((L,), jnp.float32),
        pltpu.VMEM_SHARED((NP, 16), jnp.float32),
        pltpu.SemaphoreType.DMA,
        pltpu.SemaphoreType.DMA,
    ],
)
def _sc1_pass1(src_hbm, dst_hbm, ta_hbm, tb_hbm, k_hbm, z_hbm, den_hbm,
               src_v, dst_v, sa_v, sb_v, ex_v, k_v, den_sp,
               sem0, sem1):
    c = lax.axis_index("c")
    s = lax.axis_index("s")
    wid = c * NS + s
    pltpu.sync_copy(k_hbm, k_v)
    kv = k_v[...]
    pltpu.sync_copy(z_hbm.at[pl.ds(s * ROWS, ROWS), :],
                    den_sp.at[pl.ds(s * ROWS, ROWS), :])
    plsc.subcore_barrier()

    def chunk(j, carry):
        ch = wid * CPT + j

        @pl.when(ch < NCHUNK)
        def _():
            off = ch * B
            pltpu.sync_copy(src_hbm.at[pl.ds(off, B)], src_v)
            pltpu.sync_copy(dst_hbm.at[pl.ds(off, B)], dst_v)
            ga = pltpu.async_copy(ta_hbm.at[src_v], sa_v, sem0)
            gb = pltpu.async_copy(tb_hbm.at[dst_v], sb_v, sem1)
            ga.wait()
            gb.wait()

            @plsc.parallel_loop(0, B, 1, unroll=4)
            def _(i):
                e = sa_v[i, :] + sb_v[i, :]
                e = jnp.maximum(e, 0.2 * e)
                ex_v[i, :] = jnp.exp(e - kv)

            pltpu.sync_copy(ex_v, den_sp.at[dst_v], add=True)

        return carry

    lax.fori_loop(0, CPT, chunk, 0)
    plsc.subcore_barrier()
    pltpu.sync_copy(den_sp.at[pl.ds(s * ROWS, ROWS), :],
                    den_hbm.at[c, pl.ds(s * ROWS, ROWS), :])


@functools.partial(
    pl.kernel,
    out_type=jax.ShapeDtypeStruct((NC, NP, HEADS * HID), jnp.float32),
    mesh=_mesh,
    compiler_params=_scparams,
    scratch_types=[
        pltpu.VMEM((B,), jnp.int32),
        pltpu.VMEM((B,), jnp.int32),
        pltpu.VMEM((B, 16), jnp.float32),
        pltpu.VMEM((B, 16), jnp.float32),
        pltpu.VMEM((B, 16), jnp.float32),
        pltpu.VMEM((B, HEADS * HID), jnp.float32),
        pltpu.VMEM((B, HEADS * HID), jnp.float32),
        pltpu.VMEM((L,), jnp.float32),
        pltpu.VMEM_SHARED((NP, HEADS * HID), jnp.float32),
        pltpu.SemaphoreType.DMA,
        pltpu.SemaphoreType.DMA,
        pltpu.SemaphoreType.DMA,
        pltpu.SemaphoreType.DMA,
    ],
)
def _sc1_pass2(src_hbm, dst_hbm, ta_hbm, tb_hbm, k_hbm, den_hbm, h_hbm,
               z_hbm, out_hbm,
               src_v, dst_v, sa_v, sb_v, dn_v, h_v, msg_v, k_v,
               out_sp, sem0, sem1, sem2, sem3):
    c = lax.axis_index("c")
    s = lax.axis_index("s")
    wid = c * NS + s
    pltpu.sync_copy(k_hbm, k_v)
    kv = k_v[...]
    mod8 = lax.iota(jnp.int32, L) % HEADS
    pltpu.sync_copy(z_hbm.at[pl.ds(s * ROWS, ROWS), :],
                    out_sp.at[pl.ds(s * ROWS, ROWS), :])
    plsc.subcore_barrier()

    def chunk(j, carry):
        ch = wid * CPT + j

        @pl.when(ch < NCHUNK)
        def _():
            off = ch * B
            pltpu.sync_copy(src_hbm.at[pl.ds(off, B)], src_v)
            pltpu.sync_copy(dst_hbm.at[pl.ds(off, B)], dst_v)
            ga = pltpu.async_copy(ta_hbm.at[src_v], sa_v, sem0)
            gb = pltpu.async_copy(tb_hbm.at[dst_v], sb_v, sem1)
            gd = pltpu.async_copy(den_hbm.at[dst_v], dn_v, sem2)
            gh = pltpu.async_copy(h_hbm.at[src_v], h_v, sem3)
            ga.wait()
            gb.wait()
            gd.wait()
            gh.wait()

            @plsc.parallel_loop(0, B, 1, unroll=2)
            def _(i):
                e = sa_v[i, :] + sb_v[i, :]
                e = jnp.maximum(e, 0.2 * e)
                ex = jnp.exp(e - kv)
                al = ex / (dn_v[i, :] + 1e-16)
                av = _bcast(al, mod8)
                for k in range(HEADS):
                    msg_v[i, pl.ds(k * HID, HID)] = (
                        h_v[i, pl.ds(k * HID, HID)] * av)

            pltpu.sync_copy(msg_v, out_sp.at[dst_v], add=True)

        return carry

    lax.fori_loop(0, CPT, chunk, 0)
    plsc.subcore_barrier()
    pltpu.sync_copy(out_sp.at[pl.ds(s * ROWS, ROWS), :],
                    out_hbm.at[c, pl.ds(s * ROWS, ROWS), :])


@functools.partial(
    pl.kernel,
    out_type=jax.ShapeDtypeStruct((NC, NP), jnp.float32),
    mesh=_mesh,
    compiler_params=_scparams,
    scratch_types=[
        pltpu.VMEM((B,), jnp.int32),
        pltpu.VMEM((B,), jnp.int32),
        pltpu.VMEM((B,), jnp.float32),
        pltpu.VMEM((NP,), jnp.float32),
        pltpu.VMEM((NP,), jnp.float32),
        pltpu.VMEM((L,), jnp.float32),
        pltpu.VMEM_SHARED((NP,), jnp.float32),
    ],
)
def _sc2_pass1(src_hbm, dst_hbm, as_hbm, ad_hbm, k_hbm, z_hbm, den_hbm,
               src_v, dst_v, ex_v, as_v, ad_v, k_v, den_sp):
    c = lax.axis_index("c")
    s = lax.axis_index("s")
    wid = c * NS + s
    pltpu.sync_copy(k_hbm, k_v)
    kv = k_v[...]
    pltpu.sync_copy(as_hbm, as_v)
    pltpu.sync_copy(ad_hbm, ad_v)
    pltpu.sync_copy(z_hbm.at[pl.ds(s * ROWS, ROWS)],
                    den_sp.at[pl.ds(s * ROWS, ROWS)])
    plsc.subcore_barrier()

    def chunk(j, carry):
        ch = wid * CPT + j

        @pl.when(ch < NCHUNK)
        def _():
            off = ch * B
            pltpu.sync_copy(src_hbm.at[pl.ds(off, B)], src_v)
            pltpu.sync_copy(dst_hbm.at[pl.ds(off, B)], dst_v)

            @plsc.parallel_loop(0, B // L, 1, unroll=4)
            def _(j2):
                sl = pl.ds(j2 * L, L)
                sv = plsc.load_gather(as_v, [src_v[sl]])
                dv = plsc.load_gather(ad_v, [dst_v[sl]])
                e = sv + dv
                e = jnp.maximum(e, 0.2 * e)
                ex_v[sl] = jnp.exp(e - kv)

            pltpu.sync_copy(ex_v, den_sp.at[dst_v], add=True)

        return carry

    lax.fori_loop(0, CPT, chunk, 0)
    plsc.subcore_barrier()
    pltpu.sync_copy(den_sp.at[pl.ds(s * ROWS, ROWS)],
                    den_hbm.at[c, pl.ds(s * ROWS, ROWS)])


@functools.partial(
    pl.kernel,
    out_type=jax.ShapeDtypeStruct((NC, NP, OUT), jnp.float32),
    mesh=_mesh,
    compiler_params=_scparams,
    scratch_types=[
        pltpu.VMEM((B,), jnp.int32),
        pltpu.VMEM((B,), jnp.int32),
        pltpu.VMEM((B,), jnp.float32),
        pltpu.VMEM((NP,), jnp.float32),
        pltpu.VMEM((NP,), jnp.float32),
        pltpu.VMEM((NP,), jnp.float32),
        pltpu.VMEM((B, OUT), jnp.float32),
        pltpu.VMEM((B, OUT), jnp.float32),
        pltpu.VMEM((L,), jnp.float32),
        pltpu.VMEM_SHARED((NP, OUT), jnp.float32),
        pltpu.SemaphoreType.DMA,
    ],
)
def _sc2_pass2(src_hbm, dst_hbm, as_hbm, ad_hbm, k_hbm, den_hbm, h_hbm,
               z_hbm, out_hbm,
               src_v, dst_v, al_v, as_v, ad_v, dn_v, h_v, msg_v, k_v,
               out_sp, sem0):
    c = lax.axis_index("c")
    s = lax.axis_index("s")
    wid = c * NS + s
    pltpu.sync_copy(k_hbm, k_v)
    kv = k_v[...]
    pltpu.sync_copy(as_hbm, as_v)
    pltpu.sync_copy(ad_hbm, ad_v)
    pltpu.sync_copy(den_hbm, dn_v)
    pltpu.sync_copy(z_hbm.at[pl.ds(s * ROWS, ROWS), :],
                    out_sp.at[pl.ds(s * ROWS, ROWS), :])
    plsc.subcore_barrier()

    def chunk(j, carry):
        ch = wid * CPT + j

        @pl.when(ch < NCHUNK)
        def _():
            off = ch * B
            pltpu.sync_copy(src_hbm.at[pl.ds(off, B)], src_v)
            pltpu.sync_copy(dst_hbm.at[pl.ds(off, B)], dst_v)
            gh = pltpu.async_copy(h_hbm.at[src_v], h_v, sem0)

            @plsc.parallel_loop(0, B // L, 1, unroll=4)
            def _(j2):
                sl = pl.ds(j2 * L, L)
                sv = plsc.load_gather(as_v, [src_v[sl]])
                d = dst_v[sl]
                dv = plsc.load_gather(ad_v, [d])
                e = sv + dv
                e = jnp.maximum(e, 0.2 * e)
                ex = jnp.exp(e - kv)
                den = plsc.load_gather(dn_v, [d])
                al_v[sl] = ex / (den + 1e-16)

            gh.wait()

            @plsc.parallel_loop(0, B, 1, unroll=2)
            def _(i):
                av = plsc.load_gather(al_v, [jnp.full((L,), i, jnp.int32)])
                for k in range(OUT // HID):
                    msg_v[i, pl.ds(k * HID, HID)] = (
                        h_v[i, pl.ds(k * HID, HID)] * av)

            pltpu.sync_copy(msg_v, out_sp.at[dst_v], add=True)

        return carry

    lax.fori_loop(0, CPT, chunk, 0)
    plsc.subcore_barrier()
    pltpu.sync_copy(out_sp.at[pl.ds(s * ROWS, ROWS), :],
                    out_hbm.at[c, pl.ds(s * ROWS, ROWS), :])


def kernel(x, edge_index, W1, a_src1, a_dst1, b1, W2, a_src2, a_dst2, b2):
    x_pad = jnp.pad(x, ((0, NP - N), (0, 0)))
    src = edge_index[0].astype(jnp.int32)
    dst = edge_index[1].astype(jnp.int32)

    cols = jnp.arange(HEADS * HID)
    orig = (cols % HEADS) * HID + cols // HEADS
    W1p = W1[:, orig]
    b1p = b1[orig]
    W2p = W2[orig, :]

    eye = jnp.repeat(jnp.eye(HEADS, dtype=jnp.float32), HID, axis=0)
    A_src = eye * a_src1.reshape(-1)[:, None]
    A_dst = eye * a_dst1.reshape(-1)[:, None]
    A1 = jnp.concatenate([A_src, A_dst], axis=1)[orig, :]

    h1, ta1, tb1, mx1 = _tc1(x_pad, W1p, A1)
    k1 = mx1[0, :8] + mx1[0, 8:]
    k1 = jnp.maximum(k1, 0.2 * k1)
    k16_1 = jnp.concatenate([k1, k1])

    z16 = jnp.zeros((NP, 16), jnp.float32)
    z128 = jnp.zeros((NP, HEADS * HID), jnp.float32)
    den1p = _sc1_pass1(src, dst, ta1, tb1, k16_1, z16)
    den1 = _tc_add(den1p[0], den1p[1])
    out1p = _sc1_pass2(src, dst, ta1, tb1, k16_1, den1, h1, z128)

    A2 = jnp.zeros((OUT, 16), jnp.float32)
    A2 = A2.at[:, 0].set(a_src2[0]).at[:, 1].set(a_dst2[0])
    h2, sa2, mx2 = _tc2(out1p[0], out1p[1], b1p.reshape(1, -1), W2p, A2)
    k2 = mx2[0, 0] + mx2[0, 1]
    k2 = jnp.maximum(k2, 0.2 * k2)
    k16_2 = jnp.full((L,), k2, jnp.float32)
    as2 = sa2[:, 0]
    ad2 = sa2[:, 1]

    z1d = jnp.zeros((NP,), jnp.float32)
    z64 = jnp.zeros((NP, OUT), jnp.float32)
    den2p = _sc2_pass1(src, dst, as2, ad2, k16_2, z1d)
    den2 = _tc_add(den2p[0].reshape(80, 128),
                   den2p[1].reshape(80, 128)).reshape(NP)
    out2p = _sc2_pass2(src, dst, as2, ad2, k16_2, den2, h2, z64)

    o = _tc3(out2p[0], out2p[1], b2.reshape(1, -1))
    return o[:N]
